# baseline (speedup 1.0000x reference)
import functools

import jax
import jax.numpy as jnp
from jax import lax
from jax.experimental import pallas as pl
from jax.experimental.pallas import tpu as pltpu
from jax.experimental.pallas import tpu_sc as plsc

N_NODES = 10000
N_EDGES = 320000
D_FEAT = 128
D_EDGE = 16
D_OUT = 2 * D_FEAT + D_EDGE

NUM_CORES = 1
NUM_SUBCORES = 16
NW = NUM_CORES * NUM_SUBCORES
B_PER_W = N_EDGES // NW
CHUNK = 80
N_CHUNKS = B_PER_W // CHUNK
RING = 4
N_FULL_ROUNDS = 62

_mesh = plsc.VectorSubcoreMesh(core_axis_name="c", subcore_axis_name="s",
                               num_cores=1)


@functools.partial(
    pl.kernel,
    mesh=_mesh,
    out_type=jax.ShapeDtypeStruct((N_EDGES, D_OUT), jnp.float32),
    scratch_types=(
        [pltpu.VMEM((CHUNK,), jnp.int32) for _ in range(2 * RING)]
        + [
            pltpu.VMEM((2 * CHUNK,), jnp.int32),
            pltpu.VMEM((RING, CHUNK, D_OUT), jnp.float32),
            pltpu.SemaphoreType.DMA((RING,)),
            pltpu.SemaphoreType.DMA((RING,)),
            pltpu.SemaphoreType.DMA((RING,)),
        ]
    ),
)
def _join_sc(unary, binary, index1, index2, out, *refs):
    i1s = refs[0:RING]
    i2s = refs[RING:2 * RING]
    dummy_i, row_v, isem, gsem, wsem = refs[2 * RING:]

    wid = lax.axis_index("s") * NUM_CORES + lax.axis_index("c")
    w0 = wid * B_PER_W

    def start_idx(i, s):
        base = w0 + i * CHUNK
        pltpu.async_copy(index1.at[pl.ds(base, CHUNK)], i1s[s], isem.at[s])
        pltpu.async_copy(index2.at[pl.ds(base, CHUNK)], i2s[s], isem.at[s])

    def wait_idx(s):
        pltpu.make_async_copy(index1.at[pl.ds(w0, 2 * CHUNK)], dummy_i,
                              isem.at[s]).wait()

    def start_gathers(i, b):
        pltpu.async_copy(unary.at[i1s[b]],
                         row_v.at[b, pl.ds(0, CHUNK), pl.ds(0, D_FEAT)],
                         gsem.at[b])
        pltpu.async_copy(unary.at[i2s[b]],
                         row_v.at[b, pl.ds(0, CHUNK), pl.ds(D_FEAT, D_FEAT)],
                         gsem.at[b])
        pltpu.async_copy(binary.at[pl.ds(w0 + i * CHUNK, CHUNK)],
                         row_v.at[b, pl.ds(0, CHUNK), pl.ds(2 * D_FEAT, D_EDGE)],
                         gsem.at[b])

    def drain_gathers(b):
        pltpu.make_async_copy(unary.at[i1s[b]],
                              row_v.at[b, pl.ds(0, CHUNK), pl.ds(0, D_FEAT)],
                              gsem.at[b]).wait()
        pltpu.make_async_copy(unary.at[i2s[b]],
                              row_v.at[b, pl.ds(0, CHUNK), pl.ds(D_FEAT, D_FEAT)],
                              gsem.at[b]).wait()
        pltpu.make_async_copy(binary.at[pl.ds(w0, CHUNK)],
                              row_v.at[b, pl.ds(0, CHUNK), pl.ds(2 * D_FEAT, D_EDGE)],
                              gsem.at[b]).wait()

    def start_writes(i, b):
        pltpu.async_copy(row_v.at[b],
                         out.at[pl.ds(w0 + i * CHUNK, CHUNK)],
                         wsem.at[b])

    def drain_writes(b):
        pltpu.make_async_copy(row_v.at[b],
                              out.at[pl.ds(w0, CHUNK)],
                              wsem.at[b]).wait()

    def slot(i, b, drain_w=True, idx_i=True, gather_i=True):
        sA = (b + 2) % RING
        if drain_w:
            drain_writes(sA)
        if idx_i:
            start_idx(i + 3, (b + 3) % RING)
        if gather_i:
            wait_idx(sA)
            start_gathers(i + 2, sA)
        drain_gathers(b)
        start_writes(i, b)

    start_idx(0, 0)
    start_idx(1, 1)
    start_idx(2, 2)
    wait_idx(0)
    start_gathers(0, 0)
    wait_idx(1)
    start_gathers(1, 1)

    slot(0, 0, drain_w=False)
    slot(1, 1, drain_w=False)
    slot(2, 2)
    slot(3, 3)

    def round_body(r, carry):
        i0 = r * RING
        for b in range(RING):
            slot(i0 + b, b)
        return carry

    lax.fori_loop(1, N_FULL_ROUNDS - 1, round_body, 0)

    i0 = (N_FULL_ROUNDS - 1) * RING
    slot(i0 + 0, 0)
    slot(i0 + 1, 1)
    slot(i0 + 2, 2)
    slot(i0 + 3, 3, idx_i=False)
    slot(248, 0, idx_i=False, gather_i=False)
    slot(249, 1, idx_i=False, gather_i=False)

    drain_writes(0)
    drain_writes(1)


def kernel(unary, binary, index1, index2):
    return _join_sc(unary, binary, index1, index2)


# speedup vs baseline: 1.0390x; 1.0390x over previous
import functools

import jax
import jax.numpy as jnp
from jax import lax
from jax.experimental import pallas as pl
from jax.experimental.pallas import tpu as pltpu
from jax.experimental.pallas import tpu_sc as plsc

N_NODES = 10000
N_EDGES = 320000
D_FEAT = 128
D_EDGE = 16
D_OUT = 2 * D_FEAT + D_EDGE

NUM_CORES = 2
NUM_SUBCORES = 16
NW = NUM_CORES * NUM_SUBCORES
B_PER_W = N_EDGES // NW
CHUNK = 80
N_CHUNKS = B_PER_W // CHUNK
RING = 4
N_FULL_ROUNDS = 31

_mesh = plsc.VectorSubcoreMesh(core_axis_name="c", subcore_axis_name="s")


@functools.partial(
    pl.kernel,
    mesh=_mesh,
    out_type=jax.ShapeDtypeStruct((N_EDGES, D_OUT), jnp.float32),
    scratch_types=(
        [pltpu.VMEM((CHUNK,), jnp.int32) for _ in range(2 * RING)]
        + [
            pltpu.VMEM((2 * CHUNK,), jnp.int32),
            pltpu.VMEM((RING, CHUNK, D_OUT), jnp.float32),
            pltpu.SemaphoreType.DMA((RING,)),
            pltpu.SemaphoreType.DMA((RING,)),
            pltpu.SemaphoreType.DMA((RING,)),
        ]
    ),
)
def _join_sc(unary, binary, index1, index2, out, *refs):
    i1s = refs[0:RING]
    i2s = refs[RING:2 * RING]
    dummy_i, row_v, isem, gsem, wsem = refs[2 * RING:]

    wid = lax.axis_index("s") * NUM_CORES + lax.axis_index("c")
    w0 = wid * B_PER_W

    def start_idx(i, s):
        base = w0 + i * CHUNK
        pltpu.async_copy(index1.at[pl.ds(base, CHUNK)], i1s[s], isem.at[s])
        pltpu.async_copy(index2.at[pl.ds(base, CHUNK)], i2s[s], isem.at[s])

    def wait_idx(s):
        pltpu.make_async_copy(index1.at[pl.ds(w0, 2 * CHUNK)], dummy_i,
                              isem.at[s]).wait()

    def start_gathers(i, b):
        pltpu.async_copy(unary.at[i1s[b]],
                         row_v.at[b, pl.ds(0, CHUNK), pl.ds(0, D_FEAT)],
                         gsem.at[b])
        pltpu.async_copy(unary.at[i2s[b]],
                         row_v.at[b, pl.ds(0, CHUNK), pl.ds(D_FEAT, D_FEAT)],
                         gsem.at[b])
        pltpu.async_copy(binary.at[pl.ds(w0 + i * CHUNK, CHUNK)],
                         row_v.at[b, pl.ds(0, CHUNK), pl.ds(2 * D_FEAT, D_EDGE)],
                         gsem.at[b])

    def drain_gathers(b):
        pltpu.make_async_copy(unary.at[i1s[b]],
                              row_v.at[b, pl.ds(0, CHUNK), pl.ds(0, D_FEAT)],
                              gsem.at[b]).wait()
        pltpu.make_async_copy(unary.at[i2s[b]],
                              row_v.at[b, pl.ds(0, CHUNK), pl.ds(D_FEAT, D_FEAT)],
                              gsem.at[b]).wait()
        pltpu.make_async_copy(binary.at[pl.ds(w0, CHUNK)],
                              row_v.at[b, pl.ds(0, CHUNK), pl.ds(2 * D_FEAT, D_EDGE)],
                              gsem.at[b]).wait()

    def start_writes(i, b):
        pltpu.async_copy(row_v.at[b],
                         out.at[pl.ds(w0 + i * CHUNK, CHUNK)],
                         wsem.at[b])

    def drain_writes(b):
        pltpu.make_async_copy(row_v.at[b],
                              out.at[pl.ds(w0, CHUNK)],
                              wsem.at[b]).wait()

    def slot(i, b, drain_w=True, idx_i=True, gather_i=True):
        sA = (b + 2) % RING
        if drain_w:
            drain_writes(sA)
        if idx_i:
            start_idx(i + 3, (b + 3) % RING)
        if gather_i:
            wait_idx(sA)
            start_gathers(i + 2, sA)
        drain_gathers(b)
        start_writes(i, b)

    start_idx(0, 0)
    start_idx(1, 1)
    start_idx(2, 2)
    wait_idx(0)
    start_gathers(0, 0)
    wait_idx(1)
    start_gathers(1, 1)

    slot(0, 0, drain_w=False)
    slot(1, 1, drain_w=False)
    slot(2, 2)
    slot(3, 3)

    def round_body(r, carry):
        i0 = r * RING
        for b in range(RING):
            slot(i0 + b, b)
        return carry

    lax.fori_loop(1, N_FULL_ROUNDS - 1, round_body, 0)

    i0 = (N_FULL_ROUNDS - 1) * RING
    slot(i0 + 0, 0)
    slot(i0 + 1, 1)
    slot(i0 + 2, 2, idx_i=False)
    slot(i0 + 3, 3, idx_i=False, gather_i=False)
    slot(124, 0, idx_i=False, gather_i=False)

    drain_writes(3)
    drain_writes(0)


def kernel(unary, binary, index1, index2):
    return _join_sc(unary, binary, index1, index2)
